# Initial kernel scaffold; baseline (speedup 1.0000x reference)
import jax
import jax.numpy as jnp
from jax.experimental import pallas as pl


def kernel(x, pos, edge_index, batch, W1, b1, g1, be1, W2, b2, Wg, bg, Wrel2, brel2, Wroot2, Wrel3, brel3, Wroot3):
    raise NotImplementedError("write your pallas kernel here")


# speedup vs baseline: 1.0637x; 1.0637x over previous
import functools

import jax
import jax.numpy as jnp
from jax.experimental import pallas as pl

_N = 10000
_NUM_GRAPHS = 8
_H = 128
_BLK = 1024


def _msg_body(h_ref, scale_ref, shift_ref, w2_ref, b2_ref, out_ref):
    h = h_ref[...] * scale_ref[...] + shift_ref[...]
    h = jnp.maximum(h, 0.0)
    out_ref[...] = (
        jax.lax.dot_general(h, w2_ref[...], (((1,), (0,)), ((), ())),
                            preferred_element_type=jnp.float32)
        + b2_ref[...]
    )


def _msg_stage(h, scale, shift, W2, b2):
    e_pad = h.shape[0]
    grid = e_pad // _BLK
    return pl.pallas_call(
        _msg_body,
        grid=(grid,),
        in_specs=[
            pl.BlockSpec((_BLK, _H), lambda i: (i, 0)),
            pl.BlockSpec((1, _H), lambda i: (0, 0)),
            pl.BlockSpec((1, _H), lambda i: (0, 0)),
            pl.BlockSpec((_H, _H), lambda i: (0, 0)),
            pl.BlockSpec((1, _H), lambda i: (0, 0)),
        ],
        out_specs=pl.BlockSpec((_BLK, _H), lambda i: (i, 0)),
        out_shape=jax.ShapeDtypeStruct((e_pad, _H), jnp.float32),
    )(h, scale, shift, W2, b2)


def kernel(x, pos, edge_index, batch, W1, b1, g1, be1, W2, b2, Wg, bg,
           Wrel2, brel2, Wroot2, Wrel3, brel3, Wroot3):
    n = x.shape[0]
    e = edge_index.shape[1]
    sl = jnp.arange(n, dtype=edge_index.dtype)
    src = jnp.concatenate([edge_index[0], sl])
    dst = jnp.concatenate([edge_index[1], sl])

    posp = pos @ W1[x.shape[1]:]
    A = x @ W1[: x.shape[1]] + posp + b1
    h = A[src] - posp[dst]

    mean = jnp.mean(h, axis=0)
    var = jnp.mean(h * h, axis=0) - mean * mean
    rs = jax.lax.rsqrt(var + 1e-5) * g1
    scale = rs[None]
    shift = (be1 - mean * rs)[None]

    e_tot = e + n
    e_pad = ((e_tot + _BLK - 1) // _BLK) * _BLK
    h = jnp.pad(h, ((0, e_pad - e_tot), (0, 0)))
    msg = _msg_stage(h, scale, shift, W2, b2[None])[:e_tot]

    agg = jax.ops.segment_max(msg, dst, num_segments=n)
    x1 = jax.nn.leaky_relu(agg @ Wg + bg, negative_slope=0.01)

    s2 = jax.ops.segment_sum(x1[edge_index[0]], edge_index[1], num_segments=n)
    x2 = jax.nn.leaky_relu(s2 @ Wrel2 + brel2 + x1 @ Wroot2, negative_slope=0.01)

    s3 = jax.ops.segment_sum(x2[edge_index[0]], edge_index[1], num_segments=n)
    x3 = s3 @ Wrel3 + brel3 + x2 @ Wroot3

    return jax.ops.segment_max(x3, batch, num_segments=_NUM_GRAPHS)


# speedup vs baseline: 1.6520x; 1.5531x over previous
import functools

import jax
import jax.numpy as jnp
from jax import lax
from jax.experimental import pallas as pl
from jax.experimental.pallas import tpu as pltpu
from jax.experimental.pallas import tpu_sc as plsc

_N = 10000
_NUM_GRAPHS = 8
_H = 128
_BLK = 1024

_NC, _NS, _L = 2, 16, 16
_NW = _NC * _NS

_CHUNK = 128
_SROWS = ((_N + _L + _NS * 8 - 1) // (_NS * 8)) * 8


def _conv_scatter_kernel(nchunk):
    nacc = _SROWS * _NS
    mesh = plsc.VectorSubcoreMesh(core_axis_name="c", subcore_axis_name="s",
                                  num_cores=_NC, num_subcores=_NS)

    @functools.partial(
        pl.kernel,
        out_type=jax.ShapeDtypeStruct((_NC, nacc, _H), jnp.float32),
        mesh=mesh,
        scratch_types=[
            pltpu.VMEM((_CHUNK,), jnp.int32),
            pltpu.VMEM((_CHUNK,), jnp.int32),
            pltpu.VMEM((_CHUNK,), jnp.int32),
            pltpu.VMEM((_CHUNK, _H), jnp.float32),
            pltpu.VMEM((_CHUNK, _H), jnp.float32),
            pltpu.VMEM((64, _H), jnp.float32),
            pltpu.VMEM_SHARED((nacc, _H), jnp.float32),
            pltpu.SemaphoreType.DMA,
        ],
    )
    def k(x_hbm, src_hbm, dst_hbm, out_hbm, idx0, idx1, curd,
          buf0, buf1, zero_v, acc_sh, gsem):
        c = lax.axis_index("c")
        s = lax.axis_index("s")
        wid = s * _NC + c
        ebase = wid * nchunk * _CHUNK

        def chunk_ds(i):
            return pl.ds(pl.multiple_of(ebase + i * _CHUNK, 8), _CHUNK)

        zv = jnp.zeros((_L,), jnp.float32)

        def zbody(i, _):
            for j in range(_H // _L):
                zero_v[i, pl.ds(j * _L, _L)] = zv
            return 0

        lax.fori_loop(0, 64, zbody, 0)
        nfull = _SROWS // 64
        for t in range(nfull):
            pltpu.sync_copy(zero_v, acc_sh.at[pl.ds(pl.multiple_of(s * _SROWS + t * 64, 8), 64)])
        rem = _SROWS - nfull * 64
        if rem:
            pltpu.sync_copy(zero_v.at[pl.ds(0, rem)],
                            acc_sh.at[pl.ds(pl.multiple_of(s * _SROWS + nfull * 64, 8), rem)])
        plsc.subcore_barrier()

        pltpu.sync_copy(src_hbm.at[chunk_ds(0)], idx0)
        pltpu.make_async_copy(x_hbm.at[idx0], buf0, gsem).start()

        def step(i, ti, tb, ni, nb):
            pltpu.make_async_copy(x_hbm.at[ti], tb, gsem).wait()

            @pl.when(i + 1 < nchunk)
            def _():
                pltpu.sync_copy(src_hbm.at[chunk_ds(i + 1)], ni)
                pltpu.make_async_copy(x_hbm.at[ni], nb, gsem).start()

            pltpu.sync_copy(dst_hbm.at[chunk_ds(i)], curd)
            pltpu.sync_copy(tb, acc_sh.at[curd], add=True)

        def body(i, _):
            even = lax.rem(i, 2) == 0

            @pl.when(even)
            def _():
                step(i, idx0, buf0, idx1, buf1)

            @pl.when(jnp.logical_not(even))
            def _():
                step(i, idx1, buf1, idx0, buf0)

            return 0

        lax.fori_loop(0, nchunk, body, 0)
        plsc.subcore_barrier()
        srow = pl.multiple_of(s * _SROWS, 8)
        pltpu.sync_copy(acc_sh.at[pl.ds(srow, _SROWS)],
                        out_hbm.at[c, pl.ds(srow, _SROWS)])

    return k


_OWN = 320
_NAGG = _NW * _OWN
_SCHUNK = 2048
_GB = 128
_NEG = -3.0e38


def _segmax_kernel(nschunk, e_msg):
    mesh = plsc.VectorSubcoreMesh(core_axis_name="c", subcore_axis_name="s",
                                  num_cores=_NC, num_subcores=_NS)
    aggw = (_OWN + 8) * _H

    @functools.partial(
        pl.kernel,
        out_type=jax.ShapeDtypeStruct((_NAGG * _H,), jnp.float32),
        mesh=mesh,
        scratch_types=[
            pltpu.VMEM((_SCHUNK,), jnp.int32),
            pltpu.VMEM((_SCHUNK + _GB + _L,), jnp.int32),
            pltpu.VMEM((_SCHUNK + _GB + _L,), jnp.int32),
            pltpu.VMEM((_GB, _H), jnp.float32),
            pltpu.VMEM((_GB, _H), jnp.float32),
            pltpu.VMEM((aggw,), jnp.float32),
            pltpu.SemaphoreType.DMA,
        ],
    )
    def k(msg_hbm, dst_hbm, out_hbm, dchunk, cids, ceids, mbuf0, mbuf1,
          agg, gsem):
        c = lax.axis_index("c")
        s = lax.axis_index("s")
        wid = s * _NC + c
        lo = wid * _OWN

        iota = lax.iota(jnp.int32, _L)
        negv = jnp.full((_L,), _NEG, jnp.float32)

        def initbody(i, _):
            agg[pl.ds(i * _L, _L)] = negv
            return 0

        lax.fori_loop(0, aggw // _L, initbody, 0)

        kkvecs = [kk * _L + iota for kk in range(_H // _L)]
        jidx = [jnp.full((_L, 1), j, jnp.int32) for j in range(_L)]
        dnums = lax.GatherDimensionNumbers(
            offset_dims=(), collapsed_slice_dims=(0,), start_index_map=(0,))

        def accum_rows(mbuf, base_pos):
            for g in range(_GB // _L):
                lv = cids[pl.ds(base_pos + g * _L, _L)]
                for j in range(_L):
                    lsplat = lax.gather(
                        lv, jidx[j], dnums, slice_sizes=(1,),
                        mode=lax.GatherScatterMode.PROMISE_IN_BOUNDS)
                    rbase = lsplat * _H
                    for kk in range(_H // _L):
                        idx = rbase + kkvecs[kk]
                        a = plsc.load_gather(agg, [idx])
                        m = mbuf[g * _L + j, pl.ds(kk * _L, _L)]
                        plsc.store_scatter(agg, [idx], jnp.maximum(a, m))

        def chunk_body(ci, _):
            pltpu.sync_copy(dst_hbm.at[pl.ds(pl.multiple_of(ci * _SCHUNK, 8), _SCHUNK)], dchunk)

            def scan_body(g, cnt):
                d = dchunk[pl.ds(g * _L, _L)]
                m = jnp.logical_and(d >= lo, d < lo + _OWN)
                ng = jnp.sum(jnp.where(m, 1, 0).astype(jnp.int32))

                @pl.when(ng > 0)
                def _():
                    eid = ci * _SCHUNK + g * _L + iota
                    plsc.store_compressed(cids.at[pl.ds(cnt, _L)], d - lo, mask=m)
                    plsc.store_compressed(ceids.at[pl.ds(cnt, _L)], eid, mask=m)

                return cnt + ng

            cnt = lax.fori_loop(0, _SCHUNK // _L, scan_body, 0)

            dummy = jnp.full((_L,), _OWN, jnp.int32)
            for t in range(_GB // _L):
                cids[pl.ds(cnt + t * _L, _L)] = dummy
                ceids[pl.ds(cnt + t * _L, _L)] = jnp.zeros((_L,), jnp.int32)

            nb = (cnt + _GB - 1) // _GB

            @pl.when(nb > 0)
            def _():
                pltpu.make_async_copy(
                    msg_hbm.at[ceids.at[pl.ds(0, _GB)]],
                    mbuf0, gsem).start()

                def bbody(b, _):
                    even = lax.rem(b, 2) == 0

                    def run(tb, nb_buf):
                        pltpu.make_async_copy(
                            msg_hbm.at[ceids.at[pl.ds(b * _GB, _GB)]],
                            tb, gsem).wait()

                        @pl.when(b + 1 < nb)
                        def _():
                            pltpu.make_async_copy(
                                msg_hbm.at[ceids.at[pl.ds((b + 1) * _GB, _GB)]],
                                nb_buf, gsem).start()

                        accum_rows(tb, b * _GB)

                    @pl.when(even)
                    def _():
                        run(mbuf0, mbuf1)

                    @pl.when(jnp.logical_not(even))
                    def _():
                        run(mbuf1, mbuf0)

                    return 0

                lax.fori_loop(0, nb, bbody, 0)

            return 0

        lax.fori_loop(0, nschunk, chunk_body, 0)
        pltpu.sync_copy(agg.at[pl.ds(0, _OWN * _H)],
                        out_hbm.at[pl.ds(pl.multiple_of(lo * _H, 8), _OWN * _H)])

    return k


def _segmax(msg, dstc):
    nschunk = dstc.shape[0] // _SCHUNK
    out = _segmax_kernel(nschunk, msg.shape[0])(msg, dstc)
    return out.reshape(_NAGG, _H)


def _conv_scatter(xfeat, src3, dst3):
    nchunk = src3.shape[0] // (_NW * _CHUNK)
    out = _conv_scatter_kernel(nchunk)(xfeat, src3, dst3)
    return out


def _edge_chunks(idx_src, idx_dst):
    e = idx_src.shape[0]
    ew = e // _NW
    nchunk = (ew + _CHUNK - 1) // _CHUNK
    pad = nchunk * _CHUNK - ew
    src_r = idx_src.reshape(_NW, ew)
    dst_r = idx_dst.reshape(_NW, ew)
    if pad:
        padsrc = jnp.broadcast_to((jnp.arange(pad, dtype=jnp.int32) * 37) % _N,
                                  (_NW, pad))
        paddst = jnp.broadcast_to(_N + (jnp.arange(pad, dtype=jnp.int32) % _L),
                                  (_NW, pad))
        src_r = jnp.concatenate([src_r, padsrc], axis=1)
        dst_r = jnp.concatenate([dst_r, paddst], axis=1)
    return src_r.reshape(-1), dst_r.reshape(-1)


def _msg_body(h_ref, scale_ref, shift_ref, w2_ref, b2_ref, out_ref):
    h = h_ref[...] * scale_ref[...] + shift_ref[...]
    h = jnp.maximum(h, 0.0)
    out_ref[...] = (
        jax.lax.dot_general(h, w2_ref[...], (((1,), (0,)), ((), ())),
                            preferred_element_type=jnp.float32)
        + b2_ref[...]
    )


def _msg_stage(h, scale, shift, W2, b2):
    e_pad = h.shape[0]
    grid = e_pad // _BLK
    return pl.pallas_call(
        _msg_body,
        grid=(grid,),
        in_specs=[
            pl.BlockSpec((_BLK, _H), lambda i: (i, 0)),
            pl.BlockSpec((1, _H), lambda i: (0, 0)),
            pl.BlockSpec((1, _H), lambda i: (0, 0)),
            pl.BlockSpec((_H, _H), lambda i: (0, 0)),
            pl.BlockSpec((1, _H), lambda i: (0, 0)),
        ],
        out_specs=pl.BlockSpec((_BLK, _H), lambda i: (i, 0)),
        out_shape=jax.ShapeDtypeStruct((e_pad, _H), jnp.float32),
    )(h, scale, shift, W2, b2)


_EH_NCH = 82
_EPADA = _NW * _EH_NCH * _CHUNK


def _edgeh_kernel(e_tot):
    mesh = plsc.VectorSubcoreMesh(core_axis_name="c", subcore_axis_name="s",
                                  num_cores=_NC, num_subcores=_NS)

    @functools.partial(
        pl.kernel,
        out_type=(jax.ShapeDtypeStruct((_EPADA, _H), jnp.float32),
                  jax.ShapeDtypeStruct((2 * _NW * _H,), jnp.float32)),
        mesh=mesh,
        scratch_types=[
            pltpu.VMEM((_CHUNK,), jnp.int32),
            pltpu.VMEM((_CHUNK,), jnp.int32),
            pltpu.VMEM((_CHUNK,), jnp.int32),
            pltpu.VMEM((_CHUNK,), jnp.int32),
            pltpu.VMEM((_CHUNK, _H), jnp.float32),
            pltpu.VMEM((_CHUNK, _H), jnp.float32),
            pltpu.VMEM((_CHUNK, _H), jnp.float32),
            pltpu.VMEM((_CHUNK, _H), jnp.float32),
            pltpu.VMEM((_CHUNK, _H), jnp.float32),
            pltpu.VMEM((_CHUNK, _H), jnp.float32),
            pltpu.VMEM((2 * _H,), jnp.float32),
            pltpu.SemaphoreType.DMA,
            pltpu.SemaphoreType.DMA,
        ],
    )
    def k(a_hbm, p_hbm, src_hbm, dst_hbm, h_hbm, part_hbm,
          idxs0, idxs1, idxd0, idxd1, bufa0, bufa1, bufp0, bufp1,
          hbuf0, hbuf1, psc, gsem, wsem):
        c = lax.axis_index("c")
        s = lax.axis_index("s")
        wid = s * _NC + c
        ebase = wid * _EH_NCH * _CHUNK

        def chunk_ds(i):
            return pl.ds(pl.multiple_of(ebase + i * _CHUNK, 8), _CHUNK)

        def load_idx(i, ds_, dd_):
            pltpu.sync_copy(src_hbm.at[chunk_ds(i)], ds_)
            pltpu.sync_copy(dst_hbm.at[chunk_ds(i)], dd_)

        def start_gather(ds_, dd_, ba, bp):
            pltpu.make_async_copy(a_hbm.at[ds_], ba, gsem).start()
            pltpu.make_async_copy(p_hbm.at[dd_], bp, gsem).start()

        def wait_gather(ds_, dd_, ba, bp):
            pltpu.make_async_copy(a_hbm.at[ds_], ba, gsem).wait()
            pltpu.make_async_copy(p_hbm.at[dd_], bp, gsem).wait()

        def compute(i, ba, bp, hb, carry):
            def row(r, cr):
                rowid = ebase + i * _CHUNK + r
                wf = jnp.where(rowid < e_tot, 1.0, 0.0).astype(jnp.float32)
                out = []
                for kk in range(_H // _L):
                    a = ba[r, pl.ds(kk * _L, _L)]
                    p = bp[r, pl.ds(kk * _L, _L)]
                    hv = a - p
                    hb[r, pl.ds(kk * _L, _L)] = hv
                    hw = hv * wf
                    out.append(cr[kk] + hw)
                    out.append(cr[8 + kk] + hw * hw)
                return tuple(out[0::2]) + tuple(out[1::2])

            return lax.fori_loop(0, _CHUNK, row, carry)

        def write_h(i, hb):
            pltpu.make_async_copy(hb, h_hbm.at[chunk_ds(i)], wsem).start()

        def wait_h(i, hb):
            pltpu.make_async_copy(hb, h_hbm.at[chunk_ds(i)], wsem).wait()

        load_idx(0, idxs0, idxd0)
        start_gather(idxs0, idxd0, bufa0, bufp0)
        zero16 = tuple(jnp.zeros((_L,), jnp.float32) for _ in range(16))

        def body(t, carry):
            i0 = 2 * t
            i1 = 2 * t + 1
            load_idx(i1, idxs1, idxd1)
            start_gather(idxs1, idxd1, bufa1, bufp1)
            wait_gather(idxs0, idxd0, bufa0, bufp0)

            @pl.when(t >= 1)
            def _():
                wait_h(i0 - 2, hbuf0)

            carry = compute(i0, bufa0, bufp0, hbuf0, carry)
            write_h(i0, hbuf0)

            @pl.when(i0 + 2 < _EH_NCH)
            def _():
                load_idx(i0 + 2, idxs0, idxd0)
                start_gather(idxs0, idxd0, bufa0, bufp0)

            wait_gather(idxs1, idxd1, bufa1, bufp1)

            @pl.when(t >= 1)
            def _():
                wait_h(i1 - 2, hbuf1)

            carry = compute(i1, bufa1, bufp1, hbuf1, carry)
            write_h(i1, hbuf1)
            return carry

        carry = lax.fori_loop(0, _EH_NCH // 2, body, zero16)
        wait_h(_EH_NCH - 2, hbuf0)
        wait_h(_EH_NCH - 1, hbuf1)

        for kk in range(_H // _L):
            psc[pl.ds(kk * _L, _L)] = carry[kk]
            psc[pl.ds(_H + kk * _L, _L)] = carry[8 + kk]
        pltpu.sync_copy(
            psc.at[pl.ds(0, _H)],
            part_hbm.at[pl.ds(pl.multiple_of(wid * _H, 8), _H)])
        pltpu.sync_copy(
            psc.at[pl.ds(_H, _H)],
            part_hbm.at[pl.ds(pl.multiple_of((_NW + wid) * _H, 8), _H)])

    return k


def _edgeh(A, P, srcA, dstA, e_tot):
    h, parts = _edgeh_kernel(e_tot)(A, P, srcA, dstA)
    return h, parts.reshape(2 * _NW, _H)


_NBLK = 1000


def _leaky(v):
    return jnp.where(v >= 0.0, v, 0.01 * v)


def _prep_body(x_ref, pos_ref, w1x_ref, w1p_ref, b1_ref, a_ref, p_ref):
    pp = lax.dot_general(pos_ref[...], w1p_ref[...], (((1,), (0,)), ((), ())),
                         preferred_element_type=jnp.float32)
    a_ref[...] = (
        lax.dot_general(x_ref[...], w1x_ref[...], (((1,), (0,)), ((), ())),
                        preferred_element_type=jnp.float32)
        + pp + b1_ref[...])
    p_ref[...] = pp


def _prep(x, pos, W1, b1):
    grid = _N // _NBLK
    return pl.pallas_call(
        _prep_body,
        grid=(grid,),
        in_specs=[
            pl.BlockSpec((_NBLK, _H), lambda i: (i, 0)),
            pl.BlockSpec((_NBLK, 3), lambda i: (i, 0)),
            pl.BlockSpec((_H, _H), lambda i: (0, 0)),
            pl.BlockSpec((3, _H), lambda i: (0, 0)),
            pl.BlockSpec((1, _H), lambda i: (0, 0)),
        ],
        out_specs=[pl.BlockSpec((_NBLK, _H), lambda i: (i, 0)),
                   pl.BlockSpec((_NBLK, _H), lambda i: (i, 0))],
        out_shape=[jax.ShapeDtypeStruct((_N, _H), jnp.float32),
                   jax.ShapeDtypeStruct((_N, _H), jnp.float32)],
    )(x, pos, W1[:_H], W1[_H:], b1[None])


def _bnstats_body(parts_ref, g1_ref, be1_ref, e_tot, scale_ref, shift_ref):
    sums = jnp.sum(parts_ref[0:_NW], axis=0, keepdims=True)
    sqs = jnp.sum(parts_ref[_NW:], axis=0, keepdims=True)
    mean = sums / e_tot
    var = sqs / e_tot - mean * mean
    rs = lax.rsqrt(var + 1e-5) * g1_ref[...]
    scale_ref[...] = rs
    shift_ref[...] = be1_ref[...] - mean * rs


def _bnstats(parts, g1, be1, e_tot):
    return pl.pallas_call(
        functools.partial(_bnstats_body, e_tot=float(e_tot)),
        grid=(1,),
        in_specs=[
            pl.BlockSpec((2 * _NW, _H), lambda: (0, 0)),
            pl.BlockSpec((1, _H), lambda: (0, 0)),
            pl.BlockSpec((1, _H), lambda: (0, 0)),
        ],
        out_specs=[pl.BlockSpec((1, _H), lambda: (0, 0)),
                   pl.BlockSpec((1, _H), lambda: (0, 0))],
        out_shape=[jax.ShapeDtypeStruct((1, _H), jnp.float32),
                   jax.ShapeDtypeStruct((1, _H), jnp.float32)],
    )(parts, g1[None], be1[None])


def _x1_body(a_ref, wg_ref, bg_ref, o_ref):
    o_ref[...] = _leaky(
        lax.dot_general(a_ref[...], wg_ref[...], (((1,), (0,)), ((), ())),
                        preferred_element_type=jnp.float32) + bg_ref[...])


def _x1_stage(agg, Wg, bg):
    nrow = agg.shape[0]
    return pl.pallas_call(
        _x1_body,
        grid=(nrow // _BLK,),
        in_specs=[
            pl.BlockSpec((_BLK, _H), lambda i: (i, 0)),
            pl.BlockSpec((_H, _H), lambda i: (0, 0)),
            pl.BlockSpec((1, _H), lambda i: (0, 0)),
        ],
        out_specs=pl.BlockSpec((_BLK, _H), lambda i: (i, 0)),
        out_shape=jax.ShapeDtypeStruct((nrow, _H), jnp.float32),
    )(agg, Wg, bg[None])


def _x2_body(pa_ref, pb_ref, x1_ref, wrel_ref, brel_ref, wroot_ref, o_ref,
             act):
    srow = pa_ref[...] + pb_ref[...]
    v = (lax.dot_general(srow, wrel_ref[...], (((1,), (0,)), ((), ())),
                         preferred_element_type=jnp.float32)
         + brel_ref[...]
         + lax.dot_general(x1_ref[...], wroot_ref[...],
                           (((1,), (0,)), ((), ())),
                           preferred_element_type=jnp.float32))
    o_ref[...] = _leaky(v) if act else v


def _x2_stage(pa, pb, x1, Wrel, brel, Wroot, act=True):
    nrow = x1.shape[0]
    return pl.pallas_call(
        functools.partial(_x2_body, act=act),
        grid=(nrow // _BLK,),
        in_specs=[
            pl.BlockSpec((_BLK, _H), lambda i: (i, 0)),
            pl.BlockSpec((_BLK, _H), lambda i: (i, 0)),
            pl.BlockSpec((_BLK, _H), lambda i: (i, 0)),
            pl.BlockSpec((_H, _H), lambda i: (0, 0)),
            pl.BlockSpec((1, _H), lambda i: (0, 0)),
            pl.BlockSpec((_H, _H), lambda i: (0, 0)),
        ],
        out_specs=pl.BlockSpec((_BLK, _H), lambda i: (i, 0)),
        out_shape=jax.ShapeDtypeStruct((nrow, _H), jnp.float32),
    )(pa, pb, x1, Wrel, brel[None], Wroot)


def _pool_body(x3_ref, b_ref, o_ref):
    i = pl.program_id(0)

    @pl.when(i == 0)
    def _():
        o_ref[...] = jnp.full((_NUM_GRAPHS, _H), -jnp.inf, jnp.float32)

    bvec = b_ref[0, 0, :]
    x3 = x3_ref[...]
    for g in range(_NUM_GRAPHS):
        mask = (bvec == g)[:, None]
        cand = jnp.max(jnp.where(mask, x3, -jnp.inf), axis=0)
        o_ref[pl.ds(g, 1), :] = jnp.maximum(o_ref[pl.ds(g, 1), :], cand[None])


def _pool_stage(x3, batch3):
    nrow = x3.shape[0]
    return pl.pallas_call(
        _pool_body,
        grid=(nrow // _BLK,),
        in_specs=[
            pl.BlockSpec((_BLK, _H), lambda i: (i, 0)),
            pl.BlockSpec((1, 1, _BLK), lambda i: (i, 0, 0)),
        ],
        out_specs=pl.BlockSpec((_NUM_GRAPHS, _H), lambda i: (0, 0)),
        out_shape=jax.ShapeDtypeStruct((_NUM_GRAPHS, _H), jnp.float32),
    )(x3, batch3)


def kernel(x, pos, edge_index, batch, W1, b1, g1, be1, W2, b2, Wg, bg,
           Wrel2, brel2, Wroot2, Wrel3, brel3, Wroot3):
    n = x.shape[0]
    e = edge_index.shape[1]
    sl = jnp.arange(n, dtype=edge_index.dtype)
    src = jnp.concatenate([edge_index[0], sl])
    dst = jnp.concatenate([edge_index[1], sl])

    posp = pos @ W1[x.shape[1]:]
    A = x @ W1[: x.shape[1]] + posp + b1
    h = A[src] - posp[dst]

    mean = jnp.mean(h, axis=0)
    var = jnp.mean(h * h, axis=0) - mean * mean
    rs = jax.lax.rsqrt(var + 1e-5) * g1
    scale = rs[None]
    shift = (be1 - mean * rs)[None]

    e_tot = e + n
    e_pad = ((e_tot + _BLK - 1) // _BLK) * _BLK
    h = jnp.pad(h, ((0, e_pad - e_tot), (0, 0)))
    msg = _msg_stage(h, scale, shift, W2, b2[None])

    _USE_SC_SEGMAX = False
    if _USE_SC_SEGMAX:
        e_scan = ((e_tot + _SCHUNK - 1) // _SCHUNK) * _SCHUNK
        dstc = jnp.concatenate(
            [dst, jnp.full((e_scan - e_tot,), 1 << 20, jnp.int32)])
        agg = _segmax(msg, dstc)[:n]
    else:
        agg = jax.ops.segment_max(msg[:e_tot], dst, num_segments=n)
    x1 = jax.nn.leaky_relu(agg @ Wg + bg, negative_slope=0.01)

    src3, dst3 = _edge_chunks(edge_index[0], edge_index[1])
    p2 = _conv_scatter(x1, src3, dst3)
    s2 = p2[0, :n] + p2[1, :n]
    x2 = jax.nn.leaky_relu(s2 @ Wrel2 + brel2 + x1 @ Wroot2, negative_slope=0.01)

    p3 = _conv_scatter(x2, src3, dst3)
    s3 = p3[0, :n] + p3[1, :n]
    x3 = s3 @ Wrel3 + brel3 + x2 @ Wroot3

    return jax.ops.segment_max(x3, batch, num_segments=_NUM_GRAPHS)


# speedup vs baseline: 2.6626x; 1.6117x over previous
import functools

import jax
import jax.numpy as jnp
from jax import lax
from jax.experimental import pallas as pl
from jax.experimental.pallas import tpu as pltpu
from jax.experimental.pallas import tpu_sc as plsc

_N = 10000
_NUM_GRAPHS = 8
_H = 128
_BLK = 1024

_NC, _NS, _L = 2, 16, 16
_NW = _NC * _NS

_CHUNK = 128
_SROWS = 640


def _conv_scatter_kernel(nchunk):
    nacc = _SROWS * _NS
    mesh = plsc.VectorSubcoreMesh(core_axis_name="c", subcore_axis_name="s",
                                  num_cores=_NC, num_subcores=_NS)

    @functools.partial(
        pl.kernel,
        out_type=jax.ShapeDtypeStruct((_NC, nacc, _H), jnp.float32),
        mesh=mesh,
        scratch_types=[
            pltpu.VMEM((_CHUNK,), jnp.int32),
            pltpu.VMEM((_CHUNK,), jnp.int32),
            pltpu.VMEM((_CHUNK,), jnp.int32),
            pltpu.VMEM((_CHUNK, _H), jnp.float32),
            pltpu.VMEM((_CHUNK, _H), jnp.float32),
            pltpu.VMEM((64, _H), jnp.float32),
            pltpu.VMEM_SHARED((nacc, _H), jnp.float32),
            pltpu.SemaphoreType.DMA,
        ],
    )
    def k(x_hbm, src_hbm, dst_hbm, out_hbm, idx0, idx1, curd,
          buf0, buf1, zero_v, acc_sh, gsem):
        c = lax.axis_index("c")
        s = lax.axis_index("s")
        wid = s * _NC + c
        ebase = wid * nchunk * _CHUNK

        def chunk_ds(i):
            return pl.ds(pl.multiple_of(ebase + i * _CHUNK, 8), _CHUNK)

        zv = jnp.zeros((_L,), jnp.float32)

        def zbody(i, _):
            for j in range(_H // _L):
                zero_v[i, pl.ds(j * _L, _L)] = zv
            return 0

        lax.fori_loop(0, 64, zbody, 0)
        nfull = _SROWS // 64
        for t in range(nfull):
            pltpu.sync_copy(zero_v, acc_sh.at[pl.ds(pl.multiple_of(s * _SROWS + t * 64, 8), 64)])
        rem = _SROWS - nfull * 64
        if rem:
            pltpu.sync_copy(zero_v.at[pl.ds(0, rem)],
                            acc_sh.at[pl.ds(pl.multiple_of(s * _SROWS + nfull * 64, 8), rem)])
        plsc.subcore_barrier()

        pltpu.sync_copy(src_hbm.at[chunk_ds(0)], idx0)
        pltpu.make_async_copy(x_hbm.at[idx0], buf0, gsem).start()

        def step(i, ti, tb, ni, nb):
            pltpu.make_async_copy(x_hbm.at[ti], tb, gsem).wait()

            @pl.when(i + 1 < nchunk)
            def _():
                pltpu.sync_copy(src_hbm.at[chunk_ds(i + 1)], ni)
                pltpu.make_async_copy(x_hbm.at[ni], nb, gsem).start()

            pltpu.sync_copy(dst_hbm.at[chunk_ds(i)], curd)
            pltpu.sync_copy(tb, acc_sh.at[curd], add=True)

        def body(i, _):
            even = lax.rem(i, 2) == 0

            @pl.when(even)
            def _():
                step(i, idx0, buf0, idx1, buf1)

            @pl.when(jnp.logical_not(even))
            def _():
                step(i, idx1, buf1, idx0, buf0)

            return 0

        lax.fori_loop(0, nchunk, body, 0)
        plsc.subcore_barrier()
        srow = pl.multiple_of(s * _SROWS, 8)
        pltpu.sync_copy(acc_sh.at[pl.ds(srow, _SROWS)],
                        out_hbm.at[c, pl.ds(srow, _SROWS)])

    return k


_NAGG = _SROWS * _NS


def _conv_scatter(xfeat, src3, dst3):
    nchunk = src3.shape[0] // (_NW * _CHUNK)
    out = _conv_scatter_kernel(nchunk)(xfeat, src3, dst3)
    return out


def _edge_chunks(idx_src, idx_dst):
    e = idx_src.shape[0]
    ew = e // _NW
    nchunk = (ew + _CHUNK - 1) // _CHUNK
    pad = nchunk * _CHUNK - ew
    src_r = idx_src.reshape(_NW, ew)
    dst_r = idx_dst.reshape(_NW, ew)
    if pad:
        padsrc = jnp.broadcast_to((jnp.arange(pad, dtype=jnp.int32) * 37) % _N,
                                  (_NW, pad))
        paddst = jnp.broadcast_to(_N + (jnp.arange(pad, dtype=jnp.int32) % _L),
                                  (_NW, pad))
        src_r = jnp.concatenate([src_r, padsrc], axis=1)
        dst_r = jnp.concatenate([dst_r, paddst], axis=1)
    return src_r.reshape(-1), dst_r.reshape(-1)


def _msg_body(h_ref, scale_ref, shift_ref, w2_ref, b2_ref, out_ref):
    h = h_ref[...] * scale_ref[...] + shift_ref[...]
    h = jnp.maximum(h, 0.0)
    out_ref[...] = (
        jax.lax.dot_general(h, w2_ref[...], (((1,), (0,)), ((), ())),
                            preferred_element_type=jnp.float32)
        + b2_ref[...]
    )


def _msg_stage(h, scale, shift, W2, b2):
    e_pad = h.shape[0]
    grid = e_pad // _BLK
    return pl.pallas_call(
        _msg_body,
        grid=(grid,),
        in_specs=[
            pl.BlockSpec((_BLK, _H), lambda i: (i, 0)),
            pl.BlockSpec((1, _H), lambda i: (0, 0)),
            pl.BlockSpec((1, _H), lambda i: (0, 0)),
            pl.BlockSpec((_H, _H), lambda i: (0, 0)),
            pl.BlockSpec((1, _H), lambda i: (0, 0)),
        ],
        out_specs=pl.BlockSpec((_BLK, _H), lambda i: (i, 0)),
        out_shape=jax.ShapeDtypeStruct((e_pad, _H), jnp.float32),
    )(h, scale, shift, W2, b2)


_EH_NCH = 82
_EPADA = _NW * _EH_NCH * _CHUNK


def _edgeh_kernel(e_tot):
    mesh = plsc.VectorSubcoreMesh(core_axis_name="c", subcore_axis_name="s",
                                  num_cores=_NC, num_subcores=_NS)

    @functools.partial(
        pl.kernel,
        out_type=(jax.ShapeDtypeStruct((_EPADA, _H), jnp.float32),
                  jax.ShapeDtypeStruct((2 * _NW * _H,), jnp.float32)),
        mesh=mesh,
        scratch_types=[
            pltpu.VMEM((_CHUNK,), jnp.int32),
            pltpu.VMEM((_CHUNK,), jnp.int32),
            pltpu.VMEM((_CHUNK,), jnp.int32),
            pltpu.VMEM((_CHUNK,), jnp.int32),
            pltpu.VMEM((_CHUNK, _H), jnp.float32),
            pltpu.VMEM((_CHUNK, _H), jnp.float32),
            pltpu.VMEM((_CHUNK, _H), jnp.float32),
            pltpu.VMEM((_CHUNK, _H), jnp.float32),
            pltpu.VMEM((_CHUNK, _H), jnp.float32),
            pltpu.VMEM((_CHUNK, _H), jnp.float32),
            pltpu.VMEM((2 * _H,), jnp.float32),
            pltpu.SemaphoreType.DMA,
            pltpu.SemaphoreType.DMA,
            pltpu.SemaphoreType.DMA,
            pltpu.SemaphoreType.DMA,
        ],
    )
    def k(a_hbm, p_hbm, src_hbm, dst_hbm, h_hbm, part_hbm,
          idxs0, idxs1, idxd0, idxd1, bufa0, bufa1, bufp0, bufp1,
          hbuf0, hbuf1, psc, gsem0, gsem1, wsem0, wsem1):
        c = lax.axis_index("c")
        s = lax.axis_index("s")
        wid = s * _NC + c
        ebase = wid * _EH_NCH * _CHUNK

        def chunk_ds(i):
            return pl.ds(pl.multiple_of(ebase + i * _CHUNK, 8), _CHUNK)

        def load_idx(i, ds_, dd_):
            pltpu.sync_copy(src_hbm.at[chunk_ds(i)], ds_)
            pltpu.sync_copy(dst_hbm.at[chunk_ds(i)], dd_)

        def start_gather(ds_, dd_, ba, bp, sem):
            pltpu.make_async_copy(a_hbm.at[ds_], ba, sem).start()
            pltpu.make_async_copy(p_hbm.at[dd_], bp, sem).start()

        def wait_gather(ds_, dd_, ba, bp, sem):
            pltpu.make_async_copy(a_hbm.at[ds_], ba, sem).wait()
            pltpu.make_async_copy(p_hbm.at[dd_], bp, sem).wait()

        def compute(i, ba, bp, hb, carry):
            def row(r, cr):
                rowid = ebase + i * _CHUNK + r
                wf = jnp.where(rowid < e_tot, 1.0, 0.0).astype(jnp.float32)
                out = []
                for kk in range(_H // _L):
                    a = ba[r, pl.ds(kk * _L, _L)]
                    p = bp[r, pl.ds(kk * _L, _L)]
                    hv = a - p
                    hb[r, pl.ds(kk * _L, _L)] = hv
                    hw = hv * wf
                    out.append(cr[kk] + hw)
                    out.append(cr[8 + kk] + hw * hw)
                return tuple(out[0::2]) + tuple(out[1::2])

            return lax.fori_loop(0, _CHUNK, row, carry)

        def write_h(i, hb, sem):
            pltpu.make_async_copy(hb, h_hbm.at[chunk_ds(i)], sem).start()

        def wait_h(i, hb, sem):
            pltpu.make_async_copy(hb, h_hbm.at[chunk_ds(i)], sem).wait()

        load_idx(0, idxs0, idxd0)
        start_gather(idxs0, idxd0, bufa0, bufp0, gsem0)
        zero16 = tuple(jnp.zeros((_L,), jnp.float32) for _ in range(16))

        def body(t, carry):
            i0 = 2 * t
            i1 = 2 * t + 1
            load_idx(i1, idxs1, idxd1)
            start_gather(idxs1, idxd1, bufa1, bufp1, gsem1)
            wait_gather(idxs0, idxd0, bufa0, bufp0, gsem0)

            @pl.when(t >= 1)
            def _():
                wait_h(i0 - 2, hbuf0, wsem0)

            carry = compute(i0, bufa0, bufp0, hbuf0, carry)
            write_h(i0, hbuf0, wsem0)

            @pl.when(i0 + 2 < _EH_NCH)
            def _():
                load_idx(i0 + 2, idxs0, idxd0)
                start_gather(idxs0, idxd0, bufa0, bufp0, gsem0)

            wait_gather(idxs1, idxd1, bufa1, bufp1, gsem1)

            @pl.when(t >= 1)
            def _():
                wait_h(i1 - 2, hbuf1, wsem1)

            carry = compute(i1, bufa1, bufp1, hbuf1, carry)
            write_h(i1, hbuf1, wsem1)
            return carry

        carry = lax.fori_loop(0, _EH_NCH // 2, body, zero16)
        wait_h(_EH_NCH - 2, hbuf0, wsem0)
        wait_h(_EH_NCH - 1, hbuf1, wsem1)

        for kk in range(_H // _L):
            psc[pl.ds(kk * _L, _L)] = carry[kk]
            psc[pl.ds(_H + kk * _L, _L)] = carry[8 + kk]
        pltpu.sync_copy(
            psc.at[pl.ds(0, _H)],
            part_hbm.at[pl.ds(pl.multiple_of(wid * _H, 8), _H)])
        pltpu.sync_copy(
            psc.at[pl.ds(_H, _H)],
            part_hbm.at[pl.ds(pl.multiple_of((_NW + wid) * _H, 8), _H)])

    return k


def _edgeh(A, P, srcA, dstA, e_tot):
    h, parts = _edgeh_kernel(e_tot)(A, P, srcA, dstA)
    return h, parts.reshape(2 * _NW, _H)


_NBLK = 1000


def _leaky(v):
    return jnp.where(v >= 0.0, v, 0.01 * v)


def _prep_body(x_ref, pos_ref, w1x_ref, w1p_ref, b1_ref, a_ref, p_ref):
    pp = lax.dot_general(pos_ref[...], w1p_ref[...], (((1,), (0,)), ((), ())),
                         preferred_element_type=jnp.float32)
    a_ref[...] = (
        lax.dot_general(x_ref[...], w1x_ref[...], (((1,), (0,)), ((), ())),
                        preferred_element_type=jnp.float32)
        + pp + b1_ref[...])
    p_ref[...] = pp


def _prep(x, pos, W1, b1):
    grid = _N // _NBLK
    return pl.pallas_call(
        _prep_body,
        grid=(grid,),
        in_specs=[
            pl.BlockSpec((_NBLK, _H), lambda i: (i, 0)),
            pl.BlockSpec((_NBLK, 3), lambda i: (i, 0)),
            pl.BlockSpec((_H, _H), lambda i: (0, 0)),
            pl.BlockSpec((3, _H), lambda i: (0, 0)),
            pl.BlockSpec((1, _H), lambda i: (0, 0)),
        ],
        out_specs=[pl.BlockSpec((_NBLK, _H), lambda i: (i, 0)),
                   pl.BlockSpec((_NBLK, _H), lambda i: (i, 0))],
        out_shape=[jax.ShapeDtypeStruct((_N, _H), jnp.float32),
                   jax.ShapeDtypeStruct((_N, _H), jnp.float32)],
    )(x, pos, W1[:_H], W1[_H:], b1[None])


def _bnstats_body(parts_ref, g1_ref, be1_ref, scale_ref, shift_ref, *, e_tot):
    sums = jnp.sum(parts_ref[0:_NW], axis=0, keepdims=True)
    sqs = jnp.sum(parts_ref[_NW:], axis=0, keepdims=True)
    mean = sums / e_tot
    var = sqs / e_tot - mean * mean
    rs = lax.rsqrt(var + 1e-5) * g1_ref[...]
    scale_ref[...] = rs
    shift_ref[...] = be1_ref[...] - mean * rs


def _bnstats(parts, g1, be1, e_tot):
    return pl.pallas_call(
        functools.partial(_bnstats_body, e_tot=float(e_tot)),
        grid=(1,),
        in_specs=[
            pl.BlockSpec((2 * _NW, _H), lambda i: (0, 0)),
            pl.BlockSpec((1, _H), lambda i: (0, 0)),
            pl.BlockSpec((1, _H), lambda i: (0, 0)),
        ],
        out_specs=[pl.BlockSpec((1, _H), lambda i: (0, 0)),
                   pl.BlockSpec((1, _H), lambda i: (0, 0))],
        out_shape=[jax.ShapeDtypeStruct((1, _H), jnp.float32),
                   jax.ShapeDtypeStruct((1, _H), jnp.float32)],
    )(parts, g1[None], be1[None])


def _x1_body(a_ref, wg_ref, bg_ref, o_ref):
    o_ref[...] = _leaky(
        lax.dot_general(a_ref[...], wg_ref[...], (((1,), (0,)), ((), ())),
                        preferred_element_type=jnp.float32) + bg_ref[...])


def _x1_stage(agg, Wg, bg):
    nrow = agg.shape[0]
    return pl.pallas_call(
        _x1_body,
        grid=(nrow // _BLK,),
        in_specs=[
            pl.BlockSpec((_BLK, _H), lambda i: (i, 0)),
            pl.BlockSpec((_H, _H), lambda i: (0, 0)),
            pl.BlockSpec((1, _H), lambda i: (0, 0)),
        ],
        out_specs=pl.BlockSpec((_BLK, _H), lambda i: (i, 0)),
        out_shape=jax.ShapeDtypeStruct((nrow, _H), jnp.float32),
    )(agg, Wg, bg[None])


def _x2_body(pa_ref, pb_ref, x1_ref, wrel_ref, brel_ref, wroot_ref, o_ref,
             act):
    srow = pa_ref[...] + pb_ref[...]
    v = (lax.dot_general(srow, wrel_ref[...], (((1,), (0,)), ((), ())),
                         preferred_element_type=jnp.float32)
         + brel_ref[...]
         + lax.dot_general(x1_ref[...], wroot_ref[...],
                           (((1,), (0,)), ((), ())),
                           preferred_element_type=jnp.float32))
    o_ref[...] = _leaky(v) if act else v


def _x2_stage(pa, pb, x1, Wrel, brel, Wroot, act=True):
    nrow = x1.shape[0]
    return pl.pallas_call(
        functools.partial(_x2_body, act=act),
        grid=(nrow // _BLK,),
        in_specs=[
            pl.BlockSpec((_BLK, _H), lambda i: (i, 0)),
            pl.BlockSpec((_BLK, _H), lambda i: (i, 0)),
            pl.BlockSpec((_BLK, _H), lambda i: (i, 0)),
            pl.BlockSpec((_H, _H), lambda i: (0, 0)),
            pl.BlockSpec((1, _H), lambda i: (0, 0)),
            pl.BlockSpec((_H, _H), lambda i: (0, 0)),
        ],
        out_specs=pl.BlockSpec((_BLK, _H), lambda i: (i, 0)),
        out_shape=jax.ShapeDtypeStruct((nrow, _H), jnp.float32),
    )(pa, pb, x1, Wrel, brel[None], Wroot)


def _pool_body(x3_ref, b_ref, o_ref):
    i = pl.program_id(0)

    @pl.when(i == 0)
    def _():
        o_ref[...] = jnp.full((_NUM_GRAPHS, _H), -jnp.inf, jnp.float32)

    boh = b_ref[...]
    x3 = x3_ref[...]
    for g in range(_NUM_GRAPHS):
        mask = boh[:, g:g + 1] > 0.5
        cand = jnp.max(jnp.where(mask, x3, -jnp.inf), axis=0)
        o_ref[pl.ds(g, 1), :] = jnp.maximum(o_ref[pl.ds(g, 1), :], cand[None])


def _pool_stage(x3, batch3):
    nrow = x3.shape[0]
    return pl.pallas_call(
        _pool_body,
        grid=(nrow // _BLK,),
        in_specs=[
            pl.BlockSpec((_BLK, _H), lambda i: (i, 0)),
            pl.BlockSpec((_BLK, _NUM_GRAPHS), lambda i: (i, 0)),
        ],
        out_specs=pl.BlockSpec((_NUM_GRAPHS, _H), lambda i: (0, 0)),
        out_shape=jax.ShapeDtypeStruct((_NUM_GRAPHS, _H), jnp.float32),
    )(x3, batch3)


def kernel(x, pos, edge_index, batch, W1, b1, g1, be1, W2, b2, Wg, bg,
           Wrel2, brel2, Wroot2, Wrel3, brel3, Wroot3):
    n = x.shape[0]
    e = edge_index.shape[1]
    sl = jnp.arange(n, dtype=edge_index.dtype)
    src = jnp.concatenate([edge_index[0], sl])
    dst = jnp.concatenate([edge_index[1], sl])
    e_tot = e + n

    npad = _EPADA - e_tot
    padr = (jnp.arange(npad, dtype=jnp.int32) * 37) % n
    srcA = jnp.concatenate([src, padr])
    dstA = jnp.concatenate([dst, padr])

    A, Pp = _prep(x, pos, W1, b1)
    h, parts = _edgeh(A, Pp, srcA, dstA, e_tot)

    scale, shift = _bnstats(parts, g1, be1, e_tot)
    msg = _msg_stage(h, scale, shift, W2, b2[None])

    agg0 = jax.ops.segment_max(msg[:e_tot], dst, num_segments=n)
    agg = jnp.pad(agg0, ((0, _NAGG - n), (0, 0)))

    x1 = _x1_stage(agg, Wg, bg)
    src3, dst3 = _edge_chunks(edge_index[0], edge_index[1])
    p2 = _conv_scatter(x1, src3, dst3)
    x2 = _x2_stage(p2[0], p2[1], x1, Wrel2, brel2, Wroot2, act=True)
    p3 = _conv_scatter(x2, src3, dst3)
    x3 = _x2_stage(p3[0], p3[1], x2, Wrel3, brel3, Wroot3, act=False)

    boh = (batch[:, None] == jnp.arange(_NUM_GRAPHS)[None, :]).astype(
        jnp.float32)
    boh = jnp.pad(boh, ((0, _NAGG - n), (0, 0)))
    return _pool_stage(x3, boh)
